# baseline (speedup 1.0000x reference)
import jax
import jax.numpy as jnp
from jax import lax
from jax.experimental import pallas as pl
from jax.experimental.pallas import tpu as pltpu
from jax.experimental.pallas import tpu_sc as plsc

EMB = 64
SCALE = 8.0

NW = 32
GSZ = 128
NG = 200
IDX_PER_W = NG * GSZ
TOTAL = NW * IDX_PER_W


def _emb_body(xr_hbm, table_hbm, out_hbm, idx_v, rows_v, gsem):
    c = lax.axis_index("c")
    s = lax.axis_index("s")
    wid = s * 2 + c
    base = wid * IDX_PER_W

    pltpu.sync_copy(xr_hbm.at[wid], idx_v)

    def group(g, carry):
        pltpu.async_copy(table_hbm.at[idx_v.at[g]], rows_v, gsem).wait()

        def scale4(r4, carry2):
            r = r4 * 4
            for rr in range(4):
                for cc in range(4):
                    sl = pl.ds(cc * 16, 16)
                    rows_v[r + rr, sl] = rows_v[r + rr, sl] * SCALE
            return carry2

        lax.fori_loop(0, GSZ // 4, scale4, 0)
        pltpu.sync_copy(rows_v, out_hbm.at[pl.ds(base + g * GSZ, GSZ)])
        return carry

    lax.fori_loop(0, NG, group, 0)


def kernel(x, table):
    xr = x.astype(jnp.int32).reshape(NW, NG, GSZ)
    mesh = plsc.VectorSubcoreMesh(core_axis_name="c", subcore_axis_name="s")
    out = pl.kernel(
        _emb_body,
        out_type=jax.ShapeDtypeStruct((TOTAL, EMB), jnp.float32),
        mesh=mesh,
        compiler_params=pltpu.CompilerParams(use_tc_tiling_on_sc=False),
        scratch_types=[
            pltpu.VMEM((NG, GSZ), jnp.int32),
            pltpu.VMEM((GSZ, EMB), jnp.float32),
            pltpu.SemaphoreType.DMA,
        ],
    )(xr, table)
    return out.reshape(x.shape[0], x.shape[1], EMB)


# speedup vs baseline: 1.1561x; 1.1561x over previous
import jax
import jax.numpy as jnp
from jax import lax
from jax.experimental import pallas as pl
from jax.experimental.pallas import tpu as pltpu
from jax.experimental.pallas import tpu_sc as plsc

EMB = 64
SCALE = 8.0

NW = 32
GSZ = 128
GRP = 256
NGRP = 100
NSLOT = 4
IDX_PER_W = NGRP * GRP
TOTAL = NW * IDX_PER_W
NIDXROW = IDX_PER_W // GSZ


def _emb_body(xr_hbm, table_hbm, out_hbm, idx_v, rows_v, gsem, osem):
    c = lax.axis_index("c")
    s = lax.axis_index("s")
    wid = s * 2 + c
    base = wid * IDX_PER_W

    pltpu.sync_copy(xr_hbm.at[wid], idx_v)

    def fire_gather(g, slot):
        for h in range(2):
            pltpu.make_async_copy(
                table_hbm.at[idx_v.at[2 * g + h]],
                rows_v.at[pl.ds(slot * GRP + h * GSZ, GSZ)],
                gsem.at[slot],
            ).start()

    def wait_gather(slot):
        for h in range(2):
            pltpu.make_async_copy(
                table_hbm.at[idx_v.at[h]],
                rows_v.at[pl.ds(slot * GRP + h * GSZ, GSZ)],
                gsem.at[slot],
            ).wait()

    def scale_slot(slot):
        sb = slot * GRP

        def body(i, carry):
            r = sb + i * 4
            for rr in range(4):
                for cc in range(4):
                    sl = pl.ds(cc * 16, 16)
                    rows_v[r + rr, sl] = rows_v[r + rr, sl] * SCALE
            return carry

        lax.fori_loop(0, GRP // 4, body, 0)

    def out_desc(g, slot):
        return pltpu.make_async_copy(
            rows_v.at[pl.ds(slot * GRP, GRP)],
            out_hbm.at[pl.ds(base + g * GRP, GRP)],
            osem.at[slot],
        )

    def consume(g, slot):
        wait_gather(slot)
        scale_slot(slot)
        out_desc(g, slot).start()

    fire_gather(0, 0)
    fire_gather(1, 1)

    fire_gather(2, 2)
    consume(0, 0)
    fire_gather(3, 3)
    consume(1, 1)

    def main_blk(i, carry):
        g0 = 2 + i * 4
        for db in range(4):
            g = g0 + db
            slot = (2 + db) % 4
            fslot = db
            out_desc(g, fslot).wait()
            fire_gather(g + 2, fslot)
            consume(g, slot)
        return carry

    lax.fori_loop(0, 24, main_blk, 0)

    consume(98, 2)
    consume(99, 3)

    for slot in range(4):
        out_desc(0, slot).wait()


def kernel(x, table):
    xr = x.astype(jnp.int32).reshape(NW, NIDXROW, GSZ)
    mesh = plsc.VectorSubcoreMesh(core_axis_name="c", subcore_axis_name="s")
    out = pl.kernel(
        _emb_body,
        out_type=jax.ShapeDtypeStruct((TOTAL, EMB), jnp.float32),
        mesh=mesh,
        compiler_params=pltpu.CompilerParams(use_tc_tiling_on_sc=False),
        scratch_types=[
            pltpu.VMEM((NIDXROW, GSZ), jnp.int32),
            pltpu.VMEM((NSLOT * GRP, EMB), jnp.float32),
            pltpu.SemaphoreType.DMA((NSLOT,)),
            pltpu.SemaphoreType.DMA((NSLOT,)),
        ],
    )(xr, table)
    return out.reshape(x.shape[0], x.shape[1], EMB)
